# baseline (speedup 1.0000x reference)
import functools

import jax
import jax.numpy as jnp
from jax import lax
from jax.experimental import pallas as pl
from jax.experimental.pallas import tpu as pltpu
from jax.experimental.pallas import tpu_sc as plsc

_N = 10000
_E = 320000
_D = 128
_NC = 2
_NS = 16
_NW = _NC * _NS
_EPT = _E // _NW
_K = 80
_NB = 2
_NCHUNK = 128
_EPTP = _NCHUNK * _K
_EPAD = _EPTP - _EPT
_NPAD = 10240
_RPT = _NPAD // _NS
_ZR = 128
_L = 16

_mesh = plsc.VectorSubcoreMesh(core_axis_name="c", subcore_axis_name="s")
_sc_params = pltpu.CompilerParams(needs_layout_passes=False,
                                  use_tc_tiling_on_sc=False)


@functools.partial(
    pl.kernel,
    out_type=jax.ShapeDtypeStruct((_NW, _NPAD, 1), jnp.float32),
    mesh=_mesh,
    scratch_types=[
        pltpu.VMEM((1, _EPT), jnp.int32),
        pltpu.VMEM((_NPAD, 1), jnp.float32),
    ],
    compiler_params=_sc_params,
)
def _sc_degree(dst_hbm, out_hbm, dst_v, deg_v):
    cid = lax.axis_index("c")
    sid = lax.axis_index("s")
    wid = cid * _NS + sid
    pltpu.sync_copy(dst_hbm.at[wid], dst_v)
    zf = jnp.zeros((_L,), jnp.float32)
    zi = jnp.zeros((_L,), jnp.int32)
    ramp = lax.iota(jnp.int32, _L)

    def _zero(i, c):
        plsc.store_scatter(deg_v, [ramp + i * _L, zi], zf)
        return c

    lax.fori_loop(0, _NPAD // _L, _zero, 0)
    ones = jnp.ones((_L,), jnp.float32)

    def _acc(i, c):
        idx = dst_v[0, pl.ds(i * _L, _L)]
        plsc.addupdate_scatter(deg_v, [idx, zi], ones)
        return c

    lax.fori_loop(0, _EPT // _L, _acc, 0)
    pltpu.sync_copy(deg_v, out_hbm.at[wid])


@functools.partial(
    pl.kernel,
    out_type=jax.ShapeDtypeStruct((_NC, _NPAD, _D), jnp.float32),
    mesh=_mesh,
    scratch_types=[
        pltpu.VMEM((_NCHUNK, _K), jnp.int32),
        pltpu.VMEM((_NCHUNK, _K), jnp.int32),
        pltpu.VMEM((_NB, _K, _D), jnp.float32),
        pltpu.VMEM_SHARED((_NPAD, _D), jnp.float32),
        [pltpu.SemaphoreType.DMA] * _NB,
    ],
    compiler_params=_sc_params,
)
def _sc_scatter(g_hbm, src_hbm, dst_hbm, parts_hbm,
                src_v, dst_v, rows_v, accum_s, gsem):
    cid = lax.axis_index("c")
    sid = lax.axis_index("s")
    wid = cid * _NS + sid
    pltpu.sync_copy(src_hbm.at[wid], src_v)
    pltpu.sync_copy(dst_hbm.at[wid], dst_v)

    zeros = jnp.zeros((_L,), jnp.float32)

    def _zrow(r, c):
        def _zcol(q, c2):
            rows_v[0, r, pl.ds(q * _L, _L)] = zeros
            return c2
        return lax.fori_loop(0, _D // _L, _zcol, c)

    lax.fori_loop(0, _K, _zrow, 0)
    base = pl.multiple_of(sid * _RPT, 8)

    def _zcp(t, c):
        off = pl.multiple_of(base + t * _K, 8)
        pltpu.sync_copy(rows_v.at[0], accum_s.at[pl.ds(off, _K)])
        return c

    lax.fori_loop(0, _RPT // _K, _zcp, 0)
    plsc.subcore_barrier()

    def _chunk(j, c):
        pltpu.async_copy(g_hbm.at[src_v.at[j]], rows_v.at[0], gsem[0]).wait()
        pltpu.sync_copy(rows_v.at[0], accum_s.at[dst_v.at[j]], add=True)
        return c

    lax.fori_loop(0, _NCHUNK, _chunk, 0)
    plsc.subcore_barrier()
    pltpu.sync_copy(accum_s.at[pl.ds(base, _RPT)],
                    parts_hbm.at[cid, pl.ds(base, _RPT)])


_R = 2000
_GRID = _N // _R


_PW = 8
_PSTEPS = _NW // _PW


def _pre_body(degp_ref, x_ref, w_ref, dis_ref, g_ref, acc_ref):
    j = pl.program_id(1)
    part = jnp.sum(degp_ref[...], axis=0)

    @pl.when(j == 0)
    def _():
        acc_ref[...] = part

    @pl.when(j > 0)
    def _():
        acc_ref[...] += part

    @pl.when(j == _PSTEPS - 1)
    def _():
        dis = lax.rsqrt(acc_ref[...] + 1.0)
        dis_ref[...] = dis
        h = jnp.dot(x_ref[...], w_ref[...], preferred_element_type=jnp.float32)
        g_ref[...] = h * dis


_pre_call = pl.pallas_call(
    _pre_body,
    grid=(_GRID, _PSTEPS),
    in_specs=[
        pl.BlockSpec((_PW, _R, 1), lambda i, j: (j, i, 0)),
        pl.BlockSpec((_R, _D), lambda i, j: (i, 0)),
        pl.BlockSpec((_D, _D), lambda i, j: (0, 0)),
    ],
    out_specs=[
        pl.BlockSpec((_R, 1), lambda i, j: (i, 0)),
        pl.BlockSpec((_R, _D), lambda i, j: (i, 0)),
    ],
    out_shape=[
        jax.ShapeDtypeStruct((_N, 1), jnp.float32),
        jax.ShapeDtypeStruct((_N, _D), jnp.float32),
    ],
    scratch_shapes=[pltpu.VMEM((_R, 1), jnp.float32)],
)


def _mid_body(p_ref, g_ref, dis_ref, b_ref, w_ref, o_ref):
    s = p_ref[0] + p_ref[1] + g_ref[...]
    h1 = jnp.maximum(s * dis_ref[...] + b_ref[...], 0.0)
    o_ref[...] = jnp.dot(h1, w_ref[...],
                         preferred_element_type=jnp.float32) * dis_ref[...]


_mid_call = pl.pallas_call(
    _mid_body,
    grid=(_GRID,),
    in_specs=[
        pl.BlockSpec((_NC, _R, _D), lambda i: (0, i, 0)),
        pl.BlockSpec((_R, _D), lambda i: (i, 0)),
        pl.BlockSpec((_R, 1), lambda i: (i, 0)),
        pl.BlockSpec((1, _D), lambda i: (0, 0)),
        pl.BlockSpec((_D, _D), lambda i: (0, 0)),
    ],
    out_specs=pl.BlockSpec((_R, _D), lambda i: (i, 0)),
    out_shape=jax.ShapeDtypeStruct((_N, _D), jnp.float32),
)


def _fin_body(p_ref, g_ref, dis_ref, b_ref, o_ref):
    o_ref[...] = (p_ref[0] + p_ref[1] + g_ref[...]) * dis_ref[...] + b_ref[...]


_fin_call = pl.pallas_call(
    _fin_body,
    grid=(_GRID,),
    in_specs=[
        pl.BlockSpec((_NC, _R, _D), lambda i: (0, i, 0)),
        pl.BlockSpec((_R, _D), lambda i: (i, 0)),
        pl.BlockSpec((_R, 1), lambda i: (i, 0)),
        pl.BlockSpec((1, _D), lambda i: (0, 0)),
    ],
    out_specs=pl.BlockSpec((_R, _D), lambda i: (i, 0)),
    out_shape=jax.ShapeDtypeStruct((_N, _D), jnp.float32),
)


def kernel(x, edge_index, W1, b1, W2, b2):
    src = edge_index[0]
    dst = edge_index[1]
    dst2 = dst.reshape(_NW, 1, _EPT)
    src3 = jnp.pad(src.reshape(_NW, _EPT), ((0, 0), (0, _EPAD)),
                   constant_values=0).reshape(_NW, _NCHUNK, _K)
    pad_dst = _N + (jnp.arange(_EPAD, dtype=jnp.int32) % (_NPAD - _N))
    dst3 = jnp.concatenate(
        [dst.reshape(_NW, _EPT),
         jnp.broadcast_to(pad_dst, (_NW, _EPAD))],
        axis=1).reshape(_NW, _NCHUNK, _K)
    b1r = b1.reshape(1, _D)
    b2r = b2.reshape(1, _D)

    degp = _sc_degree(dst2)
    dis, g1 = _pre_call(degp, x, W1)
    parts1 = _sc_scatter(g1, src3, dst3)
    g2 = _mid_call(parts1, g1, dis, b1r, W2)
    parts2 = _sc_scatter(g2, src3, dst3)
    return _fin_call(parts2, g2, dis, b2r)


# speedup vs baseline: 1.1628x; 1.1628x over previous
import functools

import jax
import jax.numpy as jnp
from jax import lax
from jax.experimental import pallas as pl
from jax.experimental.pallas import tpu as pltpu
from jax.experimental.pallas import tpu_sc as plsc

_N = 10000
_E = 320000
_D = 128
_NC = 2
_NS = 16
_NW = _NC * _NS
_EPT = _E // _NW
_K = 80
_NB = 2
_NCHUNK = 128
_EPTP = _NCHUNK * _K
_EPAD = _EPTP - _EPT
_NPAD = 10240
_RPT = _NPAD // _NS
_ZR = 128
_L = 16

_mesh = plsc.VectorSubcoreMesh(core_axis_name="c", subcore_axis_name="s")
_sc_params = pltpu.CompilerParams(needs_layout_passes=False,
                                  use_tc_tiling_on_sc=False)


@functools.partial(
    pl.kernel,
    out_type=jax.ShapeDtypeStruct((_NW, _NPAD, 1), jnp.float32),
    mesh=_mesh,
    scratch_types=[
        pltpu.VMEM((1, _EPT), jnp.int32),
        pltpu.VMEM((_NPAD, 1), jnp.float32),
    ],
    compiler_params=_sc_params,
)
def _sc_degree(dst_hbm, out_hbm, dst_v, deg_v):
    cid = lax.axis_index("c")
    sid = lax.axis_index("s")
    wid = cid * _NS + sid
    pltpu.sync_copy(dst_hbm.at[wid], dst_v)
    zf = jnp.zeros((_L,), jnp.float32)
    zi = jnp.zeros((_L,), jnp.int32)
    ramp = lax.iota(jnp.int32, _L)

    def _zero(i, c):
        plsc.store_scatter(deg_v, [ramp + i * _L, zi], zf)
        return c

    lax.fori_loop(0, _NPAD // _L, _zero, 0)
    ones = jnp.ones((_L,), jnp.float32)

    def _acc(i, c):
        idx = dst_v[0, pl.ds(i * _L, _L)]
        plsc.addupdate_scatter(deg_v, [idx, zi], ones)
        return c

    lax.fori_loop(0, _EPT // _L, _acc, 0)
    pltpu.sync_copy(deg_v, out_hbm.at[wid])


@functools.partial(
    pl.kernel,
    out_type=jax.ShapeDtypeStruct((_NC, _NPAD, _D), jnp.float32),
    mesh=_mesh,
    scratch_types=[
        pltpu.VMEM((_NCHUNK, _K), jnp.int32),
        pltpu.VMEM((_NCHUNK, _K), jnp.int32),
        pltpu.VMEM((_K, _D), jnp.float32),
        pltpu.VMEM((_K, _D), jnp.float32),
        pltpu.VMEM_SHARED((_NPAD, _D), jnp.float32),
        [pltpu.SemaphoreType.DMA] * _NB,
    ],
    compiler_params=_sc_params,
)
def _sc_scatter(g_hbm, src_hbm, dst_hbm, parts_hbm,
                src_v, dst_v, rows_a, rows_b, accum_s, gsem):
    cid = lax.axis_index("c")
    sid = lax.axis_index("s")
    wid = cid * _NS + sid
    pltpu.sync_copy(src_hbm.at[wid], src_v)
    pltpu.sync_copy(dst_hbm.at[wid], dst_v)

    zeros = jnp.zeros((_L,), jnp.float32)

    def _zrow(r, c):
        def _zcol(q, c2):
            rows_a[r, pl.ds(q * _L, _L)] = zeros
            return c2
        return lax.fori_loop(0, _D // _L, _zcol, c)

    lax.fori_loop(0, _K, _zrow, 0)
    base = pl.multiple_of(sid * _RPT, 8)

    def _zcp(t, c):
        off = pl.multiple_of(base + t * _K, 8)
        pltpu.sync_copy(rows_a, accum_s.at[pl.ds(off, _K)])
        return c

    lax.fori_loop(0, _RPT // _K, _zcp, 0)
    plsc.subcore_barrier()

    pltpu.async_copy(g_hbm.at[src_v.at[0]], rows_a, gsem[0])
    pltpu.async_copy(g_hbm.at[src_v.at[1]], rows_b, gsem[1])

    def _pair(t, c):
        j0 = t * 2
        pltpu.make_async_copy(g_hbm.at[pl.ds(0, _K)], rows_a, gsem[0]).wait()
        pltpu.sync_copy(rows_a, accum_s.at[dst_v.at[j0]], add=True)
        pltpu.async_copy(g_hbm.at[src_v.at[j0 + 2]], rows_a, gsem[0])
        pltpu.make_async_copy(g_hbm.at[pl.ds(0, _K)], rows_b, gsem[1]).wait()
        pltpu.sync_copy(rows_b, accum_s.at[dst_v.at[j0 + 1]], add=True)
        pltpu.async_copy(g_hbm.at[src_v.at[j0 + 3]], rows_b, gsem[1])
        return c

    lax.fori_loop(0, _NCHUNK // 2 - 1, _pair, 0)
    pltpu.make_async_copy(g_hbm.at[pl.ds(0, _K)], rows_a, gsem[0]).wait()
    pltpu.sync_copy(rows_a, accum_s.at[dst_v.at[_NCHUNK - 2]], add=True)
    pltpu.make_async_copy(g_hbm.at[pl.ds(0, _K)], rows_b, gsem[1]).wait()
    pltpu.sync_copy(rows_b, accum_s.at[dst_v.at[_NCHUNK - 1]], add=True)
    plsc.subcore_barrier()
    pltpu.sync_copy(accum_s.at[pl.ds(base, _RPT)],
                    parts_hbm.at[cid, pl.ds(base, _RPT)])


_R = 2000
_GRID = _N // _R


_PW = 8
_PSTEPS = _NW // _PW


def _pre_body(degp_ref, x_ref, w_ref, dis_ref, g_ref, acc_ref):
    j = pl.program_id(1)
    part = jnp.sum(degp_ref[...], axis=0)

    @pl.when(j == 0)
    def _():
        acc_ref[...] = part

    @pl.when(j > 0)
    def _():
        acc_ref[...] += part

    @pl.when(j == _PSTEPS - 1)
    def _():
        dis = lax.rsqrt(acc_ref[...] + 1.0)
        dis_ref[...] = dis
        h = jnp.dot(x_ref[...], w_ref[...], preferred_element_type=jnp.float32)
        g_ref[...] = h * dis


_pre_call = pl.pallas_call(
    _pre_body,
    grid=(_GRID, _PSTEPS),
    in_specs=[
        pl.BlockSpec((_PW, _R, 1), lambda i, j: (j, i, 0)),
        pl.BlockSpec((_R, _D), lambda i, j: (i, 0)),
        pl.BlockSpec((_D, _D), lambda i, j: (0, 0)),
    ],
    out_specs=[
        pl.BlockSpec((_R, 1), lambda i, j: (i, 0)),
        pl.BlockSpec((_R, _D), lambda i, j: (i, 0)),
    ],
    out_shape=[
        jax.ShapeDtypeStruct((_N, 1), jnp.float32),
        jax.ShapeDtypeStruct((_N, _D), jnp.float32),
    ],
    scratch_shapes=[pltpu.VMEM((_R, 1), jnp.float32)],
)


def _mid_body(p_ref, g_ref, dis_ref, b_ref, w_ref, o_ref):
    s = p_ref[0] + p_ref[1] + g_ref[...]
    h1 = jnp.maximum(s * dis_ref[...] + b_ref[...], 0.0)
    o_ref[...] = jnp.dot(h1, w_ref[...],
                         preferred_element_type=jnp.float32) * dis_ref[...]


_mid_call = pl.pallas_call(
    _mid_body,
    grid=(_GRID,),
    in_specs=[
        pl.BlockSpec((_NC, _R, _D), lambda i: (0, i, 0)),
        pl.BlockSpec((_R, _D), lambda i: (i, 0)),
        pl.BlockSpec((_R, 1), lambda i: (i, 0)),
        pl.BlockSpec((1, _D), lambda i: (0, 0)),
        pl.BlockSpec((_D, _D), lambda i: (0, 0)),
    ],
    out_specs=pl.BlockSpec((_R, _D), lambda i: (i, 0)),
    out_shape=jax.ShapeDtypeStruct((_N, _D), jnp.float32),
)


def _fin_body(p_ref, g_ref, dis_ref, b_ref, o_ref):
    o_ref[...] = (p_ref[0] + p_ref[1] + g_ref[...]) * dis_ref[...] + b_ref[...]


_fin_call = pl.pallas_call(
    _fin_body,
    grid=(_GRID,),
    in_specs=[
        pl.BlockSpec((_NC, _R, _D), lambda i: (0, i, 0)),
        pl.BlockSpec((_R, _D), lambda i: (i, 0)),
        pl.BlockSpec((_R, 1), lambda i: (i, 0)),
        pl.BlockSpec((1, _D), lambda i: (0, 0)),
    ],
    out_specs=pl.BlockSpec((_R, _D), lambda i: (i, 0)),
    out_shape=jax.ShapeDtypeStruct((_N, _D), jnp.float32),
)


def kernel(x, edge_index, W1, b1, W2, b2):
    src = edge_index[0]
    dst = edge_index[1]
    dst2 = dst.reshape(_NW, 1, _EPT)
    src3 = jnp.pad(src.reshape(_NW, _EPT), ((0, 0), (0, _EPAD)),
                   constant_values=0).reshape(_NW, _NCHUNK, _K)
    pad_dst = _N + (jnp.arange(_EPAD, dtype=jnp.int32) % (_NPAD - _N))
    dst3 = jnp.concatenate(
        [dst.reshape(_NW, _EPT),
         jnp.broadcast_to(pad_dst, (_NW, _EPAD))],
        axis=1).reshape(_NW, _NCHUNK, _K)
    b1r = b1.reshape(1, _D)
    b2r = b2.reshape(1, _D)

    degp = _sc_degree(dst2)
    dis, g1 = _pre_call(degp, x, W1)
    parts1 = _sc_scatter(g1, src3, dst3)
    g2 = _mid_call(parts1, g1, dis, b1r, W2)
    parts2 = _sc_scatter(g2, src3, dst3)
    return _fin_call(parts2, g2, dis, b2r)


# speedup vs baseline: 1.8908x; 1.6261x over previous
import functools

import jax
import jax.numpy as jnp
from jax import lax
from jax.experimental import pallas as pl
from jax.experimental.pallas import tpu as pltpu
from jax.experimental.pallas import tpu_sc as plsc

_N = 10000
_E = 320000
_D = 128
_NC = 2
_NS = 16
_NW = _NC * _NS
_EPT = _E // _NW
_K = 80
_NCHUNK = _EPT // _K
_NPAD = 10240
_RPT = _NPAD // _NS
_ZR = 128
_L = 16

_mesh = plsc.VectorSubcoreMesh(core_axis_name="c", subcore_axis_name="s")
_sc_params = pltpu.CompilerParams(needs_layout_passes=False,
                                  use_tc_tiling_on_sc=False)


@functools.partial(
    pl.kernel,
    out_type=jax.ShapeDtypeStruct((_NW, _NPAD, 1), jnp.float32),
    mesh=_mesh,
    scratch_types=[
        pltpu.VMEM((1, _EPT), jnp.int32),
        pltpu.VMEM((_NPAD, 1), jnp.float32),
    ],
    compiler_params=_sc_params,
)
def _sc_degree(dst_hbm, out_hbm, dst_v, deg_v):
    cid = lax.axis_index("c")
    sid = lax.axis_index("s")
    wid = cid * _NS + sid
    pltpu.sync_copy(dst_hbm.at[wid], dst_v)
    zf = jnp.zeros((_L,), jnp.float32)
    zi = jnp.zeros((_L,), jnp.int32)
    ramp = lax.iota(jnp.int32, _L)

    def _zero(i, c):
        plsc.store_scatter(deg_v, [ramp + i * _L, zi], zf)
        return c

    lax.fori_loop(0, _NPAD // _L, _zero, 0)
    ones = jnp.ones((_L,), jnp.float32)

    def _acc(i, c):
        idx = dst_v[0, pl.ds(i * _L, _L)]
        plsc.addupdate_scatter(deg_v, [idx, zi], ones)
        return c

    lax.fori_loop(0, _EPT // _L, _acc, 0)
    pltpu.sync_copy(deg_v, out_hbm.at[wid])


@functools.partial(
    pl.kernel,
    out_type=jax.ShapeDtypeStruct((_NC, _NPAD, _D), jnp.float32),
    mesh=_mesh,
    scratch_types=[
        pltpu.VMEM((_NCHUNK, _K), jnp.int32),
        pltpu.VMEM((_NCHUNK, _K), jnp.int32),
        pltpu.VMEM((_K, _D), jnp.float32),
        pltpu.VMEM((_ZR, _D), jnp.float32),
        pltpu.VMEM_SHARED((_NPAD, _D), jnp.float32),
        pltpu.SemaphoreType.DMA,
    ],
    compiler_params=_sc_params,
)
def _sc_scatter(g_hbm, src_hbm, dst_hbm, parts_hbm,
                src_v, dst_v, rows_v, zbuf_v, accum_s, sem):
    cid = lax.axis_index("c")
    sid = lax.axis_index("s")
    wid = cid * _NS + sid
    pltpu.sync_copy(src_hbm.at[wid], src_v)
    pltpu.sync_copy(dst_hbm.at[wid], dst_v)

    zeros = jnp.zeros((_L,), jnp.float32)

    def _zrow(r, c):
        def _zcol(q, c2):
            zbuf_v[r, pl.ds(q * _L, _L)] = zeros
            return c2
        return lax.fori_loop(0, _D // _L, _zcol, c)

    lax.fori_loop(0, _ZR, _zrow, 0)
    base = pl.multiple_of(sid * _RPT, 8)

    def _zcp(t, c):
        off = pl.multiple_of(base + t * _ZR, 8)
        pltpu.sync_copy(zbuf_v, accum_s.at[pl.ds(off, _ZR)])
        return c

    lax.fori_loop(0, _RPT // _ZR, _zcp, 0)
    plsc.subcore_barrier()

    def _chunk(j, c):
        pltpu.async_copy(g_hbm.at[src_v.at[j]], rows_v, sem).wait()
        pltpu.sync_copy(rows_v, accum_s.at[dst_v.at[j]], add=True)
        return c

    lax.fori_loop(0, _NCHUNK, _chunk, 0)
    plsc.subcore_barrier()
    pltpu.sync_copy(accum_s.at[pl.ds(base, _RPT)],
                    parts_hbm.at[cid, pl.ds(base, _RPT)])


_R = 2000
_GRID = _N // _R


_PW = 8
_PSTEPS = _NW // _PW


def _pre_body(degp_ref, x_ref, w_ref, dis_ref, g_ref, acc_ref):
    j = pl.program_id(1)
    part = jnp.sum(degp_ref[...], axis=0)

    @pl.when(j == 0)
    def _():
        acc_ref[...] = part

    @pl.when(j > 0)
    def _():
        acc_ref[...] += part

    @pl.when(j == _PSTEPS - 1)
    def _():
        dis = lax.rsqrt(acc_ref[...] + 1.0)
        dis_ref[...] = dis
        h = jnp.dot(x_ref[...], w_ref[...], preferred_element_type=jnp.float32)
        g_ref[...] = h * dis


_pre_call = pl.pallas_call(
    _pre_body,
    grid=(_GRID, _PSTEPS),
    in_specs=[
        pl.BlockSpec((_PW, _R, 1), lambda i, j: (j, i, 0)),
        pl.BlockSpec((_R, _D), lambda i, j: (i, 0)),
        pl.BlockSpec((_D, _D), lambda i, j: (0, 0)),
    ],
    out_specs=[
        pl.BlockSpec((_R, 1), lambda i, j: (i, 0)),
        pl.BlockSpec((_R, _D), lambda i, j: (i, 0)),
    ],
    out_shape=[
        jax.ShapeDtypeStruct((_N, 1), jnp.float32),
        jax.ShapeDtypeStruct((_N, _D), jnp.float32),
    ],
    scratch_shapes=[pltpu.VMEM((_R, 1), jnp.float32)],
)


def _mid_body(p_ref, g_ref, dis_ref, b_ref, w_ref, o_ref):
    s = p_ref[0] + p_ref[1] + g_ref[...]
    h1 = jnp.maximum(s * dis_ref[...] + b_ref[...], 0.0)
    o_ref[...] = jnp.dot(h1, w_ref[...],
                         preferred_element_type=jnp.float32) * dis_ref[...]


_mid_call = pl.pallas_call(
    _mid_body,
    grid=(_GRID,),
    in_specs=[
        pl.BlockSpec((_NC, _R, _D), lambda i: (0, i, 0)),
        pl.BlockSpec((_R, _D), lambda i: (i, 0)),
        pl.BlockSpec((_R, 1), lambda i: (i, 0)),
        pl.BlockSpec((1, _D), lambda i: (0, 0)),
        pl.BlockSpec((_D, _D), lambda i: (0, 0)),
    ],
    out_specs=pl.BlockSpec((_R, _D), lambda i: (i, 0)),
    out_shape=jax.ShapeDtypeStruct((_N, _D), jnp.float32),
)


def _fin_body(p_ref, g_ref, dis_ref, b_ref, o_ref):
    o_ref[...] = (p_ref[0] + p_ref[1] + g_ref[...]) * dis_ref[...] + b_ref[...]


_fin_call = pl.pallas_call(
    _fin_body,
    grid=(_GRID,),
    in_specs=[
        pl.BlockSpec((_NC, _R, _D), lambda i: (0, i, 0)),
        pl.BlockSpec((_R, _D), lambda i: (i, 0)),
        pl.BlockSpec((_R, 1), lambda i: (i, 0)),
        pl.BlockSpec((1, _D), lambda i: (0, 0)),
    ],
    out_specs=pl.BlockSpec((_R, _D), lambda i: (i, 0)),
    out_shape=jax.ShapeDtypeStruct((_N, _D), jnp.float32),
)


def kernel(x, edge_index, W1, b1, W2, b2):
    src = edge_index[0]
    dst = edge_index[1]
    dst2 = dst.reshape(_NW, 1, _EPT)
    src3 = src.reshape(_NW, _NCHUNK, _K)
    dst3 = dst.reshape(_NW, _NCHUNK, _K)
    b1r = b1.reshape(1, _D)
    b2r = b2.reshape(1, _D)

    degp = _sc_degree(dst2)
    dis, g1 = _pre_call(degp, x, W1)
    parts1 = _sc_scatter(g1, src3, dst3)
    g2 = _mid_call(parts1, g1, dis, b1r, W2)
    parts2 = _sc_scatter(g2, src3, dst3)
    return _fin_call(parts2, g2, dis, b2r)


# speedup vs baseline: 2.5874x; 1.3684x over previous
import functools

import jax
import jax.numpy as jnp
from jax import lax
from jax.experimental import pallas as pl
from jax.experimental.pallas import tpu as pltpu
from jax.experimental.pallas import tpu_sc as plsc

_N = 10000
_E = 320000
_D = 128
_NC = 2
_NS = 16
_NW = _NC * _NS
_EPT = _E // _NW
_K = 80
_NCHUNK = 128
_EPTP = _NCHUNK * _K
_EPAD = _EPTP - _EPT
_NPAD = 10240
_RPT = _NPAD // _NS
_ZR = 128
_L = 16

_mesh = plsc.VectorSubcoreMesh(core_axis_name="c", subcore_axis_name="s")
_sc_params = pltpu.CompilerParams(needs_layout_passes=False,
                                  use_tc_tiling_on_sc=False)


@functools.partial(
    pl.kernel,
    out_type=jax.ShapeDtypeStruct((_NW, _NPAD, 1), jnp.float32),
    mesh=_mesh,
    scratch_types=[
        pltpu.VMEM((1, _EPT), jnp.int32),
        pltpu.VMEM((_NPAD, 1), jnp.float32),
    ],
    compiler_params=_sc_params,
)
def _sc_degree(dst_hbm, out_hbm, dst_v, deg_v):
    cid = lax.axis_index("c")
    sid = lax.axis_index("s")
    wid = cid * _NS + sid
    pltpu.sync_copy(dst_hbm.at[wid], dst_v)
    zf = jnp.zeros((_L,), jnp.float32)
    zi = jnp.zeros((_L,), jnp.int32)
    ramp = lax.iota(jnp.int32, _L)

    def _zero(i, c):
        plsc.store_scatter(deg_v, [ramp + i * _L, zi], zf)
        return c

    lax.fori_loop(0, _NPAD // _L, _zero, 0)
    ones = jnp.ones((_L,), jnp.float32)

    def _acc(i, c):
        idx = dst_v[0, pl.ds(i * _L, _L)]
        plsc.addupdate_scatter(deg_v, [idx, zi], ones)
        return c

    lax.fori_loop(0, _EPT // _L, _acc, 0)
    pltpu.sync_copy(deg_v, out_hbm.at[wid])


@functools.partial(
    pl.kernel,
    out_type=jax.ShapeDtypeStruct((_NC, _NPAD, _D), jnp.float32),
    mesh=_mesh,
    scratch_types=[
        pltpu.VMEM((_NCHUNK, _K), jnp.int32),
        pltpu.VMEM((_NCHUNK, _K), jnp.int32),
        pltpu.VMEM((_K, _D), jnp.float32),
        pltpu.VMEM((_K, _D), jnp.float32),
        pltpu.VMEM_SHARED((_NPAD, _D), jnp.float32),
        pltpu.SemaphoreType.DMA,
        pltpu.SemaphoreType.DMA,
    ],
    compiler_params=_sc_params,
)
def _sc_scatter(g_hbm, src_hbm, dst_hbm, parts_hbm,
                src_v, dst_v, rows_a, rows_b, accum_s, sema, semb):
    cid = lax.axis_index("c")
    sid = lax.axis_index("s")
    wid = cid * _NS + sid
    pltpu.sync_copy(src_hbm.at[wid], src_v)
    pltpu.sync_copy(dst_hbm.at[wid], dst_v)

    zeros = jnp.zeros((_L,), jnp.float32)

    def _zrow(r, c):
        def _zcol(q, c2):
            rows_a[r, pl.ds(q * _L, _L)] = zeros
            return c2
        return lax.fori_loop(0, _D // _L, _zcol, c)

    lax.fori_loop(0, _K, _zrow, 0)
    base = pl.multiple_of(sid * _RPT, 8)

    def _zcp(t, c):
        off = pl.multiple_of(base + t * _K, 8)
        pltpu.sync_copy(rows_a, accum_s.at[pl.ds(off, _K)])
        return c

    lax.fori_loop(0, _RPT // _K, _zcp, 0)
    plsc.subcore_barrier()

    pltpu.async_copy(g_hbm.at[src_v.at[0]], rows_a, sema)
    pltpu.async_copy(g_hbm.at[src_v.at[1]], rows_b, semb)

    def _pair(t, c):
        j0 = t * 2
        pltpu.make_async_copy(g_hbm.at[pl.ds(0, _K)], rows_a, sema).wait()
        pltpu.sync_copy(rows_a, accum_s.at[dst_v.at[j0]], add=True)
        pltpu.async_copy(g_hbm.at[src_v.at[j0 + 2]], rows_a, sema)
        pltpu.make_async_copy(g_hbm.at[pl.ds(0, _K)], rows_b, semb).wait()
        pltpu.sync_copy(rows_b, accum_s.at[dst_v.at[j0 + 1]], add=True)
        pltpu.async_copy(g_hbm.at[src_v.at[j0 + 3]], rows_b, semb)
        return c

    lax.fori_loop(0, _NCHUNK // 2 - 1, _pair, 0)
    pltpu.make_async_copy(g_hbm.at[pl.ds(0, _K)], rows_a, sema).wait()
    pltpu.sync_copy(rows_a, accum_s.at[dst_v.at[_NCHUNK - 2]], add=True)
    pltpu.make_async_copy(g_hbm.at[pl.ds(0, _K)], rows_b, semb).wait()
    pltpu.sync_copy(rows_b, accum_s.at[dst_v.at[_NCHUNK - 1]], add=True)
    plsc.subcore_barrier()
    pltpu.sync_copy(accum_s.at[pl.ds(base, _RPT)],
                    parts_hbm.at[cid, pl.ds(base, _RPT)])


_R = 2000
_GRID = _N // _R


_PW = 8
_PSTEPS = _NW // _PW


def _pre_body(degp_ref, x_ref, w_ref, dis_ref, g_ref, acc_ref):
    j = pl.program_id(1)
    part = jnp.sum(degp_ref[...], axis=0)

    @pl.when(j == 0)
    def _():
        acc_ref[...] = part

    @pl.when(j > 0)
    def _():
        acc_ref[...] += part

    @pl.when(j == _PSTEPS - 1)
    def _():
        dis = lax.rsqrt(acc_ref[...] + 1.0)
        dis_ref[...] = dis
        h = jnp.dot(x_ref[...], w_ref[...], preferred_element_type=jnp.float32)
        g_ref[...] = h * dis


_pre_call = pl.pallas_call(
    _pre_body,
    grid=(_GRID, _PSTEPS),
    in_specs=[
        pl.BlockSpec((_PW, _R, 1), lambda i, j: (j, i, 0)),
        pl.BlockSpec((_R, _D), lambda i, j: (i, 0)),
        pl.BlockSpec((_D, _D), lambda i, j: (0, 0)),
    ],
    out_specs=[
        pl.BlockSpec((_R, 1), lambda i, j: (i, 0)),
        pl.BlockSpec((_R, _D), lambda i, j: (i, 0)),
    ],
    out_shape=[
        jax.ShapeDtypeStruct((_N, 1), jnp.float32),
        jax.ShapeDtypeStruct((_N, _D), jnp.float32),
    ],
    scratch_shapes=[pltpu.VMEM((_R, 1), jnp.float32)],
)


def _mid_body(p_ref, g_ref, dis_ref, b_ref, w_ref, o_ref):
    s = p_ref[0] + p_ref[1] + g_ref[...]
    h1 = jnp.maximum(s * dis_ref[...] + b_ref[...], 0.0)
    o_ref[...] = jnp.dot(h1, w_ref[...],
                         preferred_element_type=jnp.float32) * dis_ref[...]


_mid_call = pl.pallas_call(
    _mid_body,
    grid=(_GRID,),
    in_specs=[
        pl.BlockSpec((_NC, _R, _D), lambda i: (0, i, 0)),
        pl.BlockSpec((_R, _D), lambda i: (i, 0)),
        pl.BlockSpec((_R, 1), lambda i: (i, 0)),
        pl.BlockSpec((1, _D), lambda i: (0, 0)),
        pl.BlockSpec((_D, _D), lambda i: (0, 0)),
    ],
    out_specs=pl.BlockSpec((_R, _D), lambda i: (i, 0)),
    out_shape=jax.ShapeDtypeStruct((_N, _D), jnp.float32),
)


def _fin_body(p_ref, g_ref, dis_ref, b_ref, o_ref):
    o_ref[...] = (p_ref[0] + p_ref[1] + g_ref[...]) * dis_ref[...] + b_ref[...]


_fin_call = pl.pallas_call(
    _fin_body,
    grid=(_GRID,),
    in_specs=[
        pl.BlockSpec((_NC, _R, _D), lambda i: (0, i, 0)),
        pl.BlockSpec((_R, _D), lambda i: (i, 0)),
        pl.BlockSpec((_R, 1), lambda i: (i, 0)),
        pl.BlockSpec((1, _D), lambda i: (0, 0)),
    ],
    out_specs=pl.BlockSpec((_R, _D), lambda i: (i, 0)),
    out_shape=jax.ShapeDtypeStruct((_N, _D), jnp.float32),
)


def kernel(x, edge_index, W1, b1, W2, b2):
    src = edge_index[0]
    dst = edge_index[1]
    dst2 = dst.reshape(_NW, 1, _EPT)
    pad_src = (jnp.arange(_EPAD, dtype=jnp.int32) * 41) % _N
    pad_dst = _N + (jnp.arange(_EPAD, dtype=jnp.int32) % (_NPAD - _N))
    src3 = jnp.concatenate(
        [src.reshape(_NW, _EPT), jnp.broadcast_to(pad_src, (_NW, _EPAD))],
        axis=1).reshape(_NW, _NCHUNK, _K)
    dst3 = jnp.concatenate(
        [dst.reshape(_NW, _EPT), jnp.broadcast_to(pad_dst, (_NW, _EPAD))],
        axis=1).reshape(_NW, _NCHUNK, _K)
    b1r = b1.reshape(1, _D)
    b2r = b2.reshape(1, _D)

    degp = _sc_degree(dst2)
    dis, g1 = _pre_call(degp, x, W1)
    parts1 = _sc_scatter(g1, src3, dst3)
    g2 = _mid_call(parts1, g1, dis, b1r, W2)
    parts2 = _sc_scatter(g2, src3, dst3)
    return _fin_call(parts2, g2, dis, b2r)


# speedup vs baseline: 3.8186x; 1.4759x over previous
import functools

import jax
import jax.numpy as jnp
from jax import lax
from jax.experimental import pallas as pl
from jax.experimental.pallas import tpu as pltpu
from jax.experimental.pallas import tpu_sc as plsc

_N = 10000
_E = 320000
_D = 128
_NC = 2
_NS = 16
_NW = _NC * _NS
_EPT = _E // _NW
_K = 80
_NCHUNK = _EPT // _K
_NPAD = 10240
_RPT = _NPAD // _NS
_ZR = 128
_L = 16

_mesh = plsc.VectorSubcoreMesh(core_axis_name="c", subcore_axis_name="s")
_sc_params = pltpu.CompilerParams(needs_layout_passes=False,
                                  use_tc_tiling_on_sc=False)


@functools.partial(
    pl.kernel,
    out_type=jax.ShapeDtypeStruct((_NC, _NPAD, 1), jnp.float32),
    mesh=_mesh,
    scratch_types=[
        pltpu.VMEM((1, _EPT), jnp.int32),
        pltpu.VMEM((_NPAD,), jnp.float32),
        pltpu.VMEM((_NS, _RPT), jnp.float32),
        pltpu.VMEM((_RPT, 1), jnp.float32),
        pltpu.VMEM_SHARED((_NS, _NPAD), jnp.float32),
    ],
    compiler_params=_sc_params,
)
def _sc_degree(dst_hbm, out_hbm, dst_v, deg_v, red_v, col_v, stage_s):
    cid = lax.axis_index("c")
    sid = lax.axis_index("s")
    wid = cid * _NS + sid
    pltpu.sync_copy(dst_hbm.at[wid], dst_v)
    zf = jnp.zeros((_L,), jnp.float32)
    zi = jnp.zeros((_L,), jnp.int32)
    ramp = lax.iota(jnp.int32, _L)

    def _zero(i, c):
        deg_v[pl.ds(i * _L, _L)] = zf
        return c

    lax.fori_loop(0, _NPAD // _L, _zero, 0)
    ones = jnp.ones((_L,), jnp.float32)

    def _acc(i, c):
        idx = dst_v[0, pl.ds(i * _L, _L)]
        plsc.addupdate_scatter(deg_v, [idx], ones)
        return c

    lax.fori_loop(0, _EPT // _L, _acc, 0)
    pltpu.sync_copy(deg_v, stage_s.at[sid])
    plsc.subcore_barrier()
    base = pl.multiple_of(sid * _RPT, 128)
    pltpu.sync_copy(stage_s.at[:, pl.ds(base, _RPT)], red_v)

    def _red(c, carry):
        acc = red_v[0, pl.ds(c * _L, _L)]
        for r in range(1, _NS):
            acc = acc + red_v[r, pl.ds(c * _L, _L)]
        plsc.store_scatter(col_v, [ramp + c * _L, zi], acc)
        return carry

    lax.fori_loop(0, _RPT // _L, _red, 0)
    pltpu.sync_copy(col_v, out_hbm.at[cid, pl.ds(base, _RPT)])


@functools.partial(
    pl.kernel,
    out_type=jax.ShapeDtypeStruct((_NC, _NPAD, _D), jnp.float32),
    mesh=_mesh,
    scratch_types=[
        pltpu.VMEM((_NCHUNK, _K), jnp.int32),
        pltpu.VMEM((_NCHUNK, _K), jnp.int32),
        pltpu.VMEM((_K, _D), jnp.float32),
        pltpu.VMEM((_K, _D), jnp.float32),
        pltpu.VMEM_SHARED((_NPAD, _D), jnp.float32),
        pltpu.SemaphoreType.DMA,
        pltpu.SemaphoreType.DMA,
    ],
    compiler_params=_sc_params,
)
def _sc_scatter(g_hbm, src_hbm, dst_hbm, parts_hbm,
                src_v, dst_v, rows_a, rows_b, accum_s, sema, semb):
    cid = lax.axis_index("c")
    sid = lax.axis_index("s")
    wid = cid * _NS + sid
    pltpu.sync_copy(src_hbm.at[wid], src_v)
    pltpu.sync_copy(dst_hbm.at[wid], dst_v)

    zeros = jnp.zeros((_L,), jnp.float32)

    def _zrow(r, c):
        def _zcol(q, c2):
            rows_a[r, pl.ds(q * _L, _L)] = zeros
            return c2
        return lax.fori_loop(0, _D // _L, _zcol, c)

    lax.fori_loop(0, _K, _zrow, 0)
    base = pl.multiple_of(sid * _RPT, 8)

    def _zcp(t, c):
        off = pl.multiple_of(base + t * _K, 8)
        pltpu.sync_copy(rows_a, accum_s.at[pl.ds(off, _K)])
        return c

    lax.fori_loop(0, _RPT // _K, _zcp, 0)
    plsc.subcore_barrier()

    pltpu.async_copy(g_hbm.at[src_v.at[0]], rows_a, sema)
    pltpu.async_copy(g_hbm.at[src_v.at[1]], rows_b, semb)

    def _pair(t, c):
        j0 = t * 2
        pltpu.make_async_copy(g_hbm.at[pl.ds(0, _K)], rows_a, sema).wait()
        pltpu.sync_copy(rows_a, accum_s.at[dst_v.at[j0]], add=True)
        pltpu.async_copy(g_hbm.at[src_v.at[j0 + 2]], rows_a, sema)
        pltpu.make_async_copy(g_hbm.at[pl.ds(0, _K)], rows_b, semb).wait()
        pltpu.sync_copy(rows_b, accum_s.at[dst_v.at[j0 + 1]], add=True)
        pltpu.async_copy(g_hbm.at[src_v.at[j0 + 3]], rows_b, semb)
        return c

    lax.fori_loop(0, (_NCHUNK - 3) // 2, _pair, 0)
    pltpu.make_async_copy(g_hbm.at[pl.ds(0, _K)], rows_a, sema).wait()
    pltpu.sync_copy(rows_a, accum_s.at[dst_v.at[_NCHUNK - 3]], add=True)
    pltpu.async_copy(g_hbm.at[src_v.at[_NCHUNK - 1]], rows_a, sema)
    pltpu.make_async_copy(g_hbm.at[pl.ds(0, _K)], rows_b, semb).wait()
    pltpu.sync_copy(rows_b, accum_s.at[dst_v.at[_NCHUNK - 2]], add=True)
    pltpu.make_async_copy(g_hbm.at[pl.ds(0, _K)], rows_a, sema).wait()
    pltpu.sync_copy(rows_a, accum_s.at[dst_v.at[_NCHUNK - 1]], add=True)
    plsc.subcore_barrier()
    pltpu.sync_copy(accum_s.at[pl.ds(base, _RPT)],
                    parts_hbm.at[cid, pl.ds(base, _RPT)])


_R = 2000
_GRID = _N // _R


def _pre_body(degp_ref, x_ref, w_ref, dis_ref, g_ref):
    deg = degp_ref[0] + degp_ref[1] + 1.0
    dis = lax.rsqrt(deg)
    dis_ref[...] = dis
    h = jnp.dot(x_ref[...], w_ref[...], preferred_element_type=jnp.float32)
    g_ref[...] = h * dis


_pre_call = pl.pallas_call(
    _pre_body,
    grid=(_GRID,),
    in_specs=[
        pl.BlockSpec((_NC, _R, 1), lambda i: (0, i, 0)),
        pl.BlockSpec((_R, _D), lambda i: (i, 0)),
        pl.BlockSpec((_D, _D), lambda i: (0, 0)),
    ],
    out_specs=[
        pl.BlockSpec((_R, 1), lambda i: (i, 0)),
        pl.BlockSpec((_R, _D), lambda i: (i, 0)),
    ],
    out_shape=[
        jax.ShapeDtypeStruct((_N, 1), jnp.float32),
        jax.ShapeDtypeStruct((_N, _D), jnp.float32),
    ],
)


def _mid_body(p_ref, g_ref, dis_ref, b_ref, w_ref, o_ref):
    s = p_ref[0] + p_ref[1] + g_ref[...]
    h1 = jnp.maximum(s * dis_ref[...] + b_ref[...], 0.0)
    o_ref[...] = jnp.dot(h1, w_ref[...],
                         preferred_element_type=jnp.float32) * dis_ref[...]


_mid_call = pl.pallas_call(
    _mid_body,
    grid=(_GRID,),
    in_specs=[
        pl.BlockSpec((_NC, _R, _D), lambda i: (0, i, 0)),
        pl.BlockSpec((_R, _D), lambda i: (i, 0)),
        pl.BlockSpec((_R, 1), lambda i: (i, 0)),
        pl.BlockSpec((1, _D), lambda i: (0, 0)),
        pl.BlockSpec((_D, _D), lambda i: (0, 0)),
    ],
    out_specs=pl.BlockSpec((_R, _D), lambda i: (i, 0)),
    out_shape=jax.ShapeDtypeStruct((_N, _D), jnp.float32),
)


def _fin_body(p_ref, g_ref, dis_ref, b_ref, o_ref):
    o_ref[...] = (p_ref[0] + p_ref[1] + g_ref[...]) * dis_ref[...] + b_ref[...]


_fin_call = pl.pallas_call(
    _fin_body,
    grid=(_GRID,),
    in_specs=[
        pl.BlockSpec((_NC, _R, _D), lambda i: (0, i, 0)),
        pl.BlockSpec((_R, _D), lambda i: (i, 0)),
        pl.BlockSpec((_R, 1), lambda i: (i, 0)),
        pl.BlockSpec((1, _D), lambda i: (0, 0)),
    ],
    out_specs=pl.BlockSpec((_R, _D), lambda i: (i, 0)),
    out_shape=jax.ShapeDtypeStruct((_N, _D), jnp.float32),
)


def kernel(x, edge_index, W1, b1, W2, b2):
    src = edge_index[0]
    dst = edge_index[1]
    dst2 = dst.reshape(_NW, 1, _EPT)
    src3 = src.reshape(_NW, _NCHUNK, _K)
    dst3 = dst.reshape(_NW, _NCHUNK, _K)
    b1r = b1.reshape(1, _D)
    b2r = b2.reshape(1, _D)

    degp = _sc_degree(dst2)
    dis, g1 = _pre_call(degp, x, W1)
    parts1 = _sc_scatter(g1, src3, dst3)
    g2 = _mid_call(parts1, g1, dis, b1r, W2)
    parts2 = _sc_scatter(g2, src3, dst3)
    return _fin_call(parts2, g2, dis, b2r)
